# Initial kernel scaffold; baseline (speedup 1.0000x reference)
import jax
import jax.numpy as jnp
from jax import lax
from jax.experimental import pallas as pl
from jax.experimental.pallas import tpu as pltpu
from jax.experimental.pallas import tpu_sc as plsc

B, L, H, Dn, Dh = 16, 2048, 24, 256, 100
LANES = 16
HALF = L // 2
NGRP = 4
GROUP_R = (0, 1, 6, 7)
ROWS_PER_DMA = 128
NCHUNK = HALF // ROWS_PER_DMA
NDMA = NGRP * NCHUNK

MASK_SC = jnp.float32(-9999999999.0)
MASK_WC = jnp.float32(-99999999999.0)
MASK_WV = jnp.float32(-100000000000.0)


def _body(w2, l_n_h, wh_h, l_hs_h, cls_h, g_sc_h, g_wc_h,
          o_sc, o_sa, o_wn, o_wc, o_wo, o_wv,
          idx_s, gbuf, obuf, whb, clsb, lnb, lhsb, gscb, gwcb,
          scb, sab, wnb, wcb, wob, sem):
    b = lax.axis_index("s")
    h = lax.axis_index("c")
    iota = lax.iota(jnp.int32, LANES)

    pltpu.sync_copy(l_n_h, lnb)
    pltpu.sync_copy(g_wc_h, gwcb)

    base_row = (b * L + h * HALF) * 16
    for g in range(NGRP):
        r = GROUP_R[g]
        for n in range(NCHUNK):
            for m in range(ROWS_PER_DMA // LANES):
                l_vec = n * ROWS_PER_DMA + m * LANES + iota
                idx_s[g * NCHUNK + n, pl.ds(m * LANES, LANES)] = (
                    base_row + l_vec * 16 + r)

    copies = []
    for i in range(NDMA):
        copies.append(pltpu.async_copy(
            w2.at[idx_s.at[i]],
            gbuf.at[pl.ds(i * ROWS_PER_DMA, ROWS_PER_DMA)],
            sem))

    @pl.when(jnp.logical_and(b == 0, h == 0))
    def _small():
        pltpu.sync_copy(wh_h, whb)
        pltpu.sync_copy(cls_h, clsb)
        pltpu.sync_copy(l_hs_h, lhsb)
        pltpu.sync_copy(g_sc_h, gscb)
        lhs_v = lhsb[...]
        base_b = iota * (H * Dh)
        for j in range(H):
            hm = jnp.int32(j) >= lhs_v
            v0 = plsc.load_gather(whb, [base_b + (j * Dh + 0)])
            plsc.store_scatter(scb, [iota * H + j], jnp.where(hm, MASK_SC, v0))
            v8 = plsc.load_gather(whb, [base_b + (j * Dh + 8)])
            plsc.store_scatter(wcb, [iota * H + j], jnp.where(hm, MASK_WC, v8))
        gsc_v = gscb[...]
        for j in range(6):
            v = plsc.load_gather(whb, [base_b + gsc_v * Dh + (1 + j)])
            plsc.store_scatter(sab, [iota * 6 + j], v)
        for j in range(5):
            v = plsc.load_gather(clsb, [iota * Dh + j])
            plsc.store_scatter(wnb, [iota * 5 + j], v)
        for w in range(4):
            cw = plsc.load_gather(gwcb, [iota * 4 + w])
            for j in range(4):
                v = plsc.load_gather(whb, [base_b + cw * Dh + (10 + j)])
                plsc.store_scatter(wob, [iota * 16 + (w * 4 + j)], v)
        pltpu.sync_copy(scb, o_sc)
        pltpu.sync_copy(sab, o_sa)
        pltpu.sync_copy(wnb, o_wn)
        pltpu.sync_copy(wcb, o_wc)
        pltpu.sync_copy(wob, o_wo)

    for c in copies:
        c.wait()

    ln_b = plsc.load_gather(lnb, [jnp.full((LANES,), b, jnp.int32)])
    l_off = h * HALF
    for w in range(4):
        c0 = plsc.load_gather(gwcb, [jnp.full((LANES,), b * 4 + w, jnp.int32)])
        for k in range(2):
            cc = c0 if k == 0 else c0 + 4
            grp = (cc >> 4) + (0 if k == 0 else 2)
            lane = cc & 15
            row_base = grp * HALF

            def chunk(n, _, row_base=row_base, lane=lane, w=w, k=k):
                l_vec = n * LANES + iota
                vals = plsc.load_gather(gbuf, [row_base + l_vec, lane])
                nm = (l_off + l_vec) >= ln_b
                vals = jnp.where(nm, MASK_WV, vals)
                plsc.store_scatter(
                    obuf, [w * HALF + l_vec, jnp.full((LANES,), k, jnp.int32)],
                    vals)
                return 0

            lax.fori_loop(0, HALF // LANES, chunk, 0)

    for w in range(4):
        pltpu.sync_copy(obuf.at[pl.ds(w * HALF, HALF)],
                        o_wv.at[b * 4 + w, pl.ds(h * HALF, HALF)])


def _sc_call(w2, l_n, wh, l_hs, cls, g_sc, g_wc):
    return pl.kernel(
        _body,
        out_type=[
            jax.ShapeDtypeStruct((B * H,), jnp.float32),
            jax.ShapeDtypeStruct((B * 6,), jnp.float32),
            jax.ShapeDtypeStruct((B * 5,), jnp.float32),
            jax.ShapeDtypeStruct((B * H,), jnp.float32),
            jax.ShapeDtypeStruct((B * 16,), jnp.float32),
            jax.ShapeDtypeStruct((B * 4, L, 2), jnp.float32),
        ],
        mesh=plsc.VectorSubcoreMesh(core_axis_name="c", subcore_axis_name="s"),
        scratch_types=[
            pltpu.VMEM((NDMA, ROWS_PER_DMA), jnp.int32),
            pltpu.VMEM((NGRP * HALF, 16), jnp.float32),
            pltpu.VMEM((4 * HALF, 2), jnp.float32),
            pltpu.VMEM((B * H * Dh,), jnp.float32),
            pltpu.VMEM((B * Dh,), jnp.float32),
            pltpu.VMEM((B,), jnp.int32),
            pltpu.VMEM((B,), jnp.int32),
            pltpu.VMEM((B,), jnp.int32),
            pltpu.VMEM((B * 4,), jnp.int32),
            pltpu.VMEM((B * H,), jnp.float32),
            pltpu.VMEM((B * 6,), jnp.float32),
            pltpu.VMEM((B * 5,), jnp.float32),
            pltpu.VMEM((B * H,), jnp.float32),
            pltpu.VMEM((B * 16,), jnp.float32),
            pltpu.SemaphoreType.DMA,
        ],
    )(w2, l_n, wh, l_hs, cls, g_sc, g_wc)


def kernel(wemb_n, l_n, wemb_h, l_hs, cls_vec, g_sc, g_sa, g_wn, g_wc, g_wo):
    w2 = wemb_n.reshape(B * L * (Dn // 16), 16)
    o_sc, o_sa, o_wn, o_wc, o_wo, o_wv = _sc_call(
        w2,
        l_n.astype(jnp.int32),
        wemb_h.reshape(B * H * Dh),
        l_hs.astype(jnp.int32),
        cls_vec.reshape(B * Dh),
        g_sc.astype(jnp.int32),
        g_wc.reshape(B * 4).astype(jnp.int32),
    )
    return (o_sc.reshape(B, H), o_sa.reshape(B, 6), o_wn.reshape(B, 5),
            o_wc.reshape(B, H), o_wo.reshape(B, 4, 4),
            o_wv.reshape(B, 4, L, 2))


# speedup vs baseline: 1.1912x; 1.1912x over previous
import jax
import jax.numpy as jnp
from jax import lax
from jax.experimental import pallas as pl
from jax.experimental.pallas import tpu as pltpu
from jax.experimental.pallas import tpu_sc as plsc

B, L, H, Dn, Dh = 16, 2048, 24, 256, 100
LANES = 16
HALF = L // 2
NGRP = 4
GROUP_R = (0, 1, 6, 7)
ROWS_PER_DMA = 128
NCHUNK = HALF // ROWS_PER_DMA
NDMA = NGRP * NCHUNK

MASK_SC = -9999999999.0
MASK_WC = -99999999999.0
MASK_WV = -100000000000.0


def _body(w2, l_n_h, wh_h, l_hs_h, cls_h, g_sc_h, g_wc_h,
          o_sc, o_sa, o_wn, o_wc, o_wo, o_wv,
          idx_s, gbuf, obuf, whb, clsb, lnb, lhsb, gscb, gwcb,
          scb, sab, wnb, wcb, wob, sem):
    b = lax.axis_index("s")
    h = lax.axis_index("c")
    iota = lax.iota(jnp.int32, LANES)

    pltpu.sync_copy(l_n_h, lnb)
    pltpu.sync_copy(g_wc_h, gwcb)

    base_row = (b * L + h * HALF) * 16
    for g in range(NGRP):
        r = GROUP_R[g]
        for n in range(NCHUNK):
            for m in range(ROWS_PER_DMA // LANES):
                l_vec = n * ROWS_PER_DMA + m * LANES + iota
                idx_s[g * NCHUNK + n, pl.ds(m * LANES, LANES)] = (
                    base_row + l_vec * 16 + r)

    copies = []
    for i in range(NDMA):
        copies.append(pltpu.async_copy(
            w2.at[idx_s.at[i]],
            gbuf.at[pl.ds(i * ROWS_PER_DMA, ROWS_PER_DMA)],
            sem))

    @pl.when(jnp.logical_and(b == 0, h == 0))
    def _small():
        pltpu.sync_copy(wh_h.at[pl.ds(0, B * H), pl.ds(0, 16)], whb)
        pltpu.sync_copy(cls_h.at[pl.ds(0, B), pl.ds(0, 16)], clsb)
        pltpu.sync_copy(l_hs_h, lhsb)
        pltpu.sync_copy(g_sc_h, gscb)
        lhs_v = lhsb[...]
        rows_b = iota * H
        for j in range(H):
            hm = jnp.int32(j) >= lhs_v
            v0 = plsc.load_gather(whb, [rows_b + j, jnp.full((LANES,), 0, jnp.int32)])
            plsc.store_scatter(scb, [iota * H + j], jnp.where(hm, MASK_SC, v0))
            v8 = plsc.load_gather(whb, [rows_b + j, jnp.full((LANES,), 8, jnp.int32)])
            plsc.store_scatter(wcb, [iota * H + j], jnp.where(hm, MASK_WC, v8))
        gsc_v = gscb[...]
        for j in range(6):
            v = plsc.load_gather(whb, [rows_b + gsc_v, jnp.full((LANES,), 1 + j, jnp.int32)])
            plsc.store_scatter(sab, [iota * 6 + j], v)
        for j in range(5):
            v = plsc.load_gather(clsb, [iota, jnp.full((LANES,), j, jnp.int32)])
            plsc.store_scatter(wnb, [iota * 5 + j], v)
        for w in range(4):
            cw = plsc.load_gather(gwcb, [iota * 4 + w])
            for j in range(4):
                v = plsc.load_gather(whb, [rows_b + cw, jnp.full((LANES,), 10 + j, jnp.int32)])
                plsc.store_scatter(wob, [iota * 16 + (w * 4 + j)], v)
        pltpu.sync_copy(scb, o_sc)
        pltpu.sync_copy(sab, o_sa)
        pltpu.sync_copy(wnb, o_wn)
        pltpu.sync_copy(wcb, o_wc)
        pltpu.sync_copy(wob, o_wo)

    for c in copies:
        c.wait()

    ln_b = plsc.load_gather(lnb, [jnp.full((LANES,), b, jnp.int32)])
    l_off = h * HALF
    for w in range(4):
        c0 = plsc.load_gather(gwcb, [jnp.full((LANES,), b * 4 + w, jnp.int32)])
        for k in range(2):
            cc = c0 if k == 0 else c0 + 4
            grp = (cc >> 4) + (0 if k == 0 else 2)
            lane = cc & 15
            row_base = grp * HALF

            def chunk(n, _, row_base=row_base, lane=lane, w=w, k=k):
                l_vec = n * LANES + iota
                vals = plsc.load_gather(gbuf, [row_base + l_vec, lane])
                nm = (l_off + l_vec) >= ln_b
                vals = jnp.where(nm, MASK_WV, vals)
                plsc.store_scatter(
                    obuf, [w * HALF + l_vec, jnp.full((LANES,), k, jnp.int32)],
                    vals)
                return 0

            lax.fori_loop(0, HALF // LANES, chunk, 0)

    for w in range(4):
        pltpu.sync_copy(obuf.at[pl.ds(w * HALF, HALF)],
                        o_wv.at[b * 4 + w, pl.ds(h * HALF, HALF)])


def _sc_call(w2, l_n, wh, l_hs, cls, g_sc, g_wc):
    return pl.kernel(
        _body,
        out_type=[
            jax.ShapeDtypeStruct((B * H,), jnp.float32),
            jax.ShapeDtypeStruct((B * 6,), jnp.float32),
            jax.ShapeDtypeStruct((B * 5,), jnp.float32),
            jax.ShapeDtypeStruct((B * H,), jnp.float32),
            jax.ShapeDtypeStruct((B * 16,), jnp.float32),
            jax.ShapeDtypeStruct((B * 4, L, 2), jnp.float32),
        ],
        mesh=plsc.VectorSubcoreMesh(core_axis_name="c", subcore_axis_name="s"),
        compiler_params=pltpu.CompilerParams(
            needs_layout_passes=False, use_tc_tiling_on_sc=False),
        scratch_types=[
            pltpu.VMEM((NDMA, ROWS_PER_DMA), jnp.int32),
            pltpu.VMEM((NGRP * HALF, 16), jnp.float32),
            pltpu.VMEM((4 * HALF, 2), jnp.float32),
            pltpu.VMEM((B * H, 16), jnp.float32),
            pltpu.VMEM((B, 16), jnp.float32),
            pltpu.VMEM((B,), jnp.int32),
            pltpu.VMEM((B,), jnp.int32),
            pltpu.VMEM((B,), jnp.int32),
            pltpu.VMEM((B * 4,), jnp.int32),
            pltpu.VMEM((B * H,), jnp.float32),
            pltpu.VMEM((B * 6,), jnp.float32),
            pltpu.VMEM((B * 5,), jnp.float32),
            pltpu.VMEM((B * H,), jnp.float32),
            pltpu.VMEM((B * 16,), jnp.float32),
            pltpu.SemaphoreType.DMA,
        ],
    )(w2, l_n, wh, l_hs, cls, g_sc, g_wc)


def kernel(wemb_n, l_n, wemb_h, l_hs, cls_vec, g_sc, g_sa, g_wn, g_wc, g_wo):
    w2 = wemb_n.reshape(B * L * (Dn // 16), 16)
    o_sc, o_sa, o_wn, o_wc, o_wo, o_wv = _sc_call(
        w2,
        l_n.astype(jnp.int32),
        wemb_h.reshape(B * H, Dh),
        l_hs.astype(jnp.int32),
        cls_vec,
        g_sc.astype(jnp.int32),
        g_wc.reshape(B * 4).astype(jnp.int32),
    )
    return (o_sc.reshape(B, H), o_sa.reshape(B, 6), o_wn.reshape(B, 5),
            o_wc.reshape(B, H), o_wo.reshape(B, 4, 4),
            o_wv.reshape(B, 4, L, 2))
